# Initial kernel scaffold; baseline (speedup 1.0000x reference)
import jax
import jax.numpy as jnp
from jax.experimental import pallas as pl


def kernel(inputs, bb):
    raise NotImplementedError("write your pallas kernel here")


# speedup vs baseline: 12.2590x; 12.2590x over previous
import numpy as np
import jax
import jax.numpy as jnp
from jax.experimental import pallas as pl
from jax.experimental.pallas import tpu as pltpu

_B, _H, _QQ, _KK = 8, 8, 1024, 1024
_G = 32
_QBLK = 256
_NQS = _QQ // _QBLK


def _make_selector() -> np.ndarray:
    qx = np.arange(_G)
    kx = np.arange(_G)
    dmat = np.abs(kx[None, :] - qx[:, None])
    m = (np.arange(_G)[:, None, None] == dmat[None, :, :]).astype(np.float32)
    return m.reshape(_G, _G * _G)


_M_SEL = _make_selector()


def _w_body(bbR_ref, m_ref, w_ref):
    t = bbR_ref[0]
    w_ref[0] = jnp.dot(t, m_ref[...], preferred_element_type=jnp.float32)


def _add_body(w_ref, in_ref, out_ref, pos_ref):
    b = pl.program_id(1)
    qs = pl.program_id(2)

    @pl.when((b == 0) & (qs == 0))
    def _build_bias():
        for qy in range(_G):
            for ky in range(_G):
                d = abs(ky - qy)
                pos_ref[qy * _G:(qy + 1) * _G, ky * _G:(ky + 1) * _G] = w_ref[0, d]

    out_ref[0, 0] = in_ref[0, 0] + pos_ref[pl.ds(qs * _QBLK, _QBLK), :]


@jax.jit
def kernel(inputs, bb):
    bbR = jnp.transpose(bb).reshape(_H, _G, _G)
    m = jnp.asarray(_M_SEL)

    w_flat = pl.pallas_call(
        _w_body,
        grid=(_H,),
        in_specs=[
            pl.BlockSpec((1, _G, _G), lambda h: (h, 0, 0)),
            pl.BlockSpec((_G, _G * _G), lambda h: (0, 0)),
        ],
        out_specs=pl.BlockSpec((1, _G, _G * _G), lambda h: (h, 0, 0)),
        out_shape=jax.ShapeDtypeStruct((_H, _G, _G * _G), jnp.float32),
    )(bbR, m)
    w4 = w_flat.reshape(_H, _G, _G, _G)

    return pl.pallas_call(
        _add_body,
        grid=(_H, _B, _NQS),
        in_specs=[
            pl.BlockSpec((1, _G, _G, _G), lambda h, b, qs: (h, 0, 0, 0)),
            pl.BlockSpec((1, 1, _QBLK, _KK), lambda h, b, qs: (b, h, qs, 0)),
        ],
        out_specs=pl.BlockSpec((1, 1, _QBLK, _KK), lambda h, b, qs: (b, h, qs, 0)),
        out_shape=jax.ShapeDtypeStruct((_B, _H, _QQ, _KK), jnp.float32),
        scratch_shapes=[pltpu.VMEM((_QQ, _KK), jnp.float32)],
    )(w4, inputs)


# speedup vs baseline: 18.7883x; 1.5326x over previous
import numpy as np
import jax
import jax.numpy as jnp
from jax.experimental import pallas as pl
from jax.experimental.pallas import tpu as pltpu

_B, _H, _QQ, _KK = 8, 8, 1024, 1024
_G = 32
_QBLK = 256
_NQS = _QQ // _QBLK


def _make_selector() -> np.ndarray:
    qx = np.arange(_G)
    kx = np.arange(_G)
    dmat = np.abs(kx[None, :] - qx[:, None])
    m = (np.arange(_G)[:, None, None] == dmat[None, :, :]).astype(np.float32)
    return m.reshape(_G, _G * _G)


_M_SEL = _make_selector()


def _w_body(bbR_ref, m_ref, w_ref):
    t = bbR_ref[0]
    w_ref[0] = jnp.dot(t, m_ref[...], preferred_element_type=jnp.float32)


def _add_body(w_ref, in_ref, out_ref, pos_ref):
    b = pl.program_id(1)

    @pl.when(b == 0)
    def _build_bias():
        for ky in range(_G):
            pos_ref[0:_G, ky * _G:(ky + 1) * _G] = w_ref[0, ky]
        for qy in range(1, _G):
            r = qy * _G
            pos_ref[r:r + _G, _G:] = pos_ref[r - _G:r, :_KK - _G]
            pos_ref[r:r + _G, 0:_G] = w_ref[0, qy]

    out_ref[0, 0] = in_ref[0, 0] + pos_ref[...]


@jax.jit
def kernel(inputs, bb):
    bbR = jnp.transpose(bb).reshape(_H, _G, _G)
    m = jnp.asarray(_M_SEL)

    w_flat = pl.pallas_call(
        _w_body,
        grid=(_H,),
        in_specs=[
            pl.BlockSpec((1, _G, _G), lambda h: (h, 0, 0)),
            pl.BlockSpec((_G, _G * _G), lambda h: (0, 0)),
        ],
        out_specs=pl.BlockSpec((1, _G, _G * _G), lambda h: (h, 0, 0)),
        out_shape=jax.ShapeDtypeStruct((_H, _G, _G * _G), jnp.float32),
    )(bbR, m)
    w4 = w_flat.reshape(_H, _G, _G, _G)

    return pl.pallas_call(
        _add_body,
        grid=(_H, _B),
        in_specs=[
            pl.BlockSpec((1, _G, _G, _G), lambda h, b: (h, 0, 0, 0)),
            pl.BlockSpec((1, 1, _QQ, _KK), lambda h, b: (b, h, 0, 0)),
        ],
        out_specs=pl.BlockSpec((1, 1, _QQ, _KK), lambda h, b: (b, h, 0, 0)),
        out_shape=jax.ShapeDtypeStruct((_B, _H, _QQ, _KK), jnp.float32),
        scratch_shapes=[pltpu.VMEM((_QQ, _KK), jnp.float32)],
    )(w4, inputs)


# speedup vs baseline: 20.2331x; 1.0769x over previous
import numpy as np
import jax
import jax.numpy as jnp
from jax.experimental import pallas as pl
from jax.experimental.pallas import tpu as pltpu

_B, _H, _QQ, _KK = 8, 8, 1024, 1024
_G = 32
_QBLK = 256
_NQS = _QQ // _QBLK


def _make_selector() -> np.ndarray:
    qx = np.arange(_G)
    kx = np.arange(_G)
    dmat = np.abs(kx[None, :] - qx[:, None])
    m = (np.arange(_G)[:, None, None] == dmat[None, :, :]).astype(np.float32)
    return m.reshape(_G, _G * _G)


_M_SEL = _make_selector()


def _w_body(bbR_ref, m_ref, w_ref):
    t = bbR_ref[0]
    w_ref[0] = jnp.dot(t, m_ref[...], preferred_element_type=jnp.float32)


def _add_body(w_ref, in_ref, out_ref, pos_ref):
    b = pl.program_id(1)

    @pl.when(b == 0)
    def _build_bias():
        for ky in range(_G):
            pos_ref[0:_G, ky * _G:(ky + 1) * _G] = w_ref[0, ky]
        for qy in range(1, _G):
            r = qy * _G
            pos_ref[r:r + _G, _G:] = pos_ref[r - _G:r, :_KK - _G]
            pos_ref[r:r + _G, 0:_G] = w_ref[0, qy]

    pos = pos_ref[...]
    out_ref[0, 0] = in_ref[0, 0] + pos
    out_ref[1, 0] = in_ref[1, 0] + pos


@jax.jit
def kernel(inputs, bb):
    bbR = jnp.transpose(bb).reshape(_H, _G, _G)
    m = jnp.asarray(_M_SEL)

    w_flat = pl.pallas_call(
        _w_body,
        grid=(_H,),
        in_specs=[
            pl.BlockSpec((1, _G, _G), lambda h: (h, 0, 0)),
            pl.BlockSpec((_G, _G * _G), lambda h: (0, 0)),
        ],
        out_specs=pl.BlockSpec((1, _G, _G * _G), lambda h: (h, 0, 0)),
        out_shape=jax.ShapeDtypeStruct((_H, _G, _G * _G), jnp.float32),
    )(bbR, m)
    w4 = w_flat.reshape(_H, _G, _G, _G)

    return pl.pallas_call(
        _add_body,
        grid=(_H, _B // 2),
        in_specs=[
            pl.BlockSpec((1, _G, _G, _G), lambda h, b: (h, 0, 0, 0)),
            pl.BlockSpec((2, 1, _QQ, _KK), lambda h, b: (b, h, 0, 0)),
        ],
        out_specs=pl.BlockSpec((2, 1, _QQ, _KK), lambda h, b: (b, h, 0, 0)),
        out_shape=jax.ShapeDtypeStruct((_B, _H, _QQ, _KK), jnp.float32),
        scratch_shapes=[pltpu.VMEM((_QQ, _KK), jnp.float32)],
    )(w4, inputs)
